# baseline (speedup 1.0000x reference)
import functools

import jax
import jax.numpy as jnp
from jax import lax
from jax.experimental import pallas as pl
from jax.experimental.pallas import tpu as pltpu
from jax.experimental.pallas import tpu_sc as plsc

N = 10000
E = 320000
D_IN = 128
D_ID = 16
D_EF = 16
D_MSG = 128
D_H = 256
HD = D_H // 2

NC = 2
NS = 16
LANES = 16
NW = NC * NS

SLICES = (192000, 128000)
ROWS_PER_TILE = 624
TAIL_ROWS = N - NS * ROWS_PER_TILE
ZR = 48


def _chunk_size(epw):
    for k in range(128, 0, -8):
        if epw % k == 0:
            return k
    raise ValueError(epw)


@functools.lru_cache(maxsize=None)
def _get_mesh():
    return plsc.VectorSubcoreMesh(core_axis_name="c", subcore_axis_name="s",
                                  num_cores=NC, num_subcores=NS)


def _make_gather_body(epw, kg, nchunk):
    nbuf = 3

    def body(pd_hbm, ps_hbm, ei_hbm, ej_hbm, pre_hbm,
             idxi_all, idxj_all, a0, b0, a1, b1, a2, b2,
             gsa0, gsb0, gsa1, gsb1, gsa2, gsb2, ss0, ss1, ss2):
        wid = lax.axis_index("s") * NC + lax.axis_index("c")
        base0 = wid * epw
        pltpu.sync_copy(ei_hbm.at[pl.ds(base0, epw)], idxi_all)
        pltpu.sync_copy(ej_hbm.at[pl.ds(base0, epw)], idxj_all)

        sets = ((a0, b0, gsa0, gsb0, ss0),
                (a1, b1, gsa1, gsb1, ss1),
                (a2, b2, gsa2, gsb2, ss2))

        def issue(setidx, it):
            a, b, gsa, gsb, _ = sets[setidx]
            off = it * kg
            pltpu.async_copy(pd_hbm.at[idxi_all.at[pl.ds(off, kg)]], a, gsa)
            pltpu.async_copy(ps_hbm.at[idxj_all.at[pl.ds(off, kg)]], b, gsb)

        def process(setidx, it, issue_next):
            a, b, gsa, gsb, ss = sets[setidx]
            nset = (setidx + 1) % nbuf
            na = sets[nset][0]
            nss = sets[nset][4]
            pltpu.make_async_copy(
                pd_hbm.at[idxi_all.at[pl.ds(0, kg)]], a, gsa).wait()
            pltpu.make_async_copy(
                ps_hbm.at[idxj_all.at[pl.ds(0, kg)]], b, gsb).wait()
            if issue_next:
                @pl.when(it >= nbuf - 1)
                def _wait_next_store():
                    pltpu.make_async_copy(
                        na, pre_hbm.at[pl.ds(base0, kg)], nss).wait()
                issue(nset, it + 1)

            def row(r, carry):
                for cc in range(HD // LANES):
                    sl = pl.ds(cc * LANES, LANES)
                    av = plsc.bitcast(a[r, sl], jnp.bfloat16)
                    bv = plsc.bitcast(b[r, sl], jnp.bfloat16)
                    a[r, sl] = plsc.bitcast(av + bv, jnp.float32)
                return carry

            lax.fori_loop(0, kg, row, 0, unroll=False)
            pltpu.async_copy(a, pre_hbm.at[pl.ds(base0 + it * kg, kg)], ss)

        issue(0, 0)
        ntriple = (nchunk - 2) // nbuf

        def body3(k, carry):
            process(0, 3 * k, True)
            process(1, 3 * k + 1, True)
            process(2, 3 * k + 2, True)
            return carry

        lax.fori_loop(0, ntriple, body3, 0, unroll=False)
        for it in range(nbuf * ntriple, nchunk):
            process(it % nbuf, it, it + 1 < nchunk)
        for a, _b, _gsa, _gsb, ss in sets:
            pltpu.make_async_copy(a, pre_hbm.at[pl.ds(base0, kg)], ss).wait()

    return body


@functools.lru_cache(maxsize=None)
def _build_gather(e_cnt):
    epw = e_cnt // NW
    kg = _chunk_size(epw)
    nchunk = epw // kg
    return pl.kernel(
        _make_gather_body(epw, kg, nchunk),
        out_type=jax.ShapeDtypeStruct((e_cnt, HD), jnp.float32),
        mesh=_get_mesh(),
        compiler_params=pltpu.CompilerParams(needs_layout_passes=False),
        scratch_types=(
            [pltpu.VMEM((epw,), jnp.int32)] * 2
            + [pltpu.VMEM((kg, HD), jnp.float32)] * 6
            + [pltpu.SemaphoreType.DMA] * 9
        ),
    )


def _sc_gather(pd, ps, ei, ej):
    return _build_gather(ei.shape[0])(pd, ps, ei, ej)


def _make_scatter_body(phases):
    def body(*refs):
        nph = len(phases)
        msgs_refs = refs[0:2 * nph:2]
        idx_hbm_refs = refs[1:2 * nph:2]
        agg_hbm = refs[2 * nph]
        idx_v_refs = refs[2 * nph + 1:3 * nph + 1]
        m0, m1, z_v, acc_sh, sm0, sm1, sa0, sa1, zsem = refs[3 * nph + 1:]
        c = lax.axis_index("c")
        s = lax.axis_index("s")
        wid = c * NS + s
        sets = ((m0, sm0, sa0), (m1, sm1, sa1))

        for ih, iv in zip(idx_hbm_refs, idx_v_refs):
            pltpu.sync_copy(ih.at[wid], iv)
        zero = jnp.zeros((LANES,), jnp.float32)

        def zrow(r, carry):
            for cc in range(D_MSG // LANES):
                z_v[r, pl.ds(cc * LANES, LANES)] = zero
            return carry

        lax.fori_loop(0, ZR, zrow, 0, unroll=False)
        row0 = s * ROWS_PER_TILE
        zcopies = [
            pltpu.async_copy(z_v, acc_sh.at[pl.ds(row0 + k * ZR, ZR)], zsem)
            for k in range(ROWS_PER_TILE // ZR)
        ]

        @pl.when(s == NS - 1)
        def _zero_tail():
            pltpu.async_copy(z_v.at[pl.ds(0, TAIL_ROWS)],
                             acc_sh.at[pl.ds(NS * ROWS_PER_TILE, TAIL_ROWS)],
                             zsem).wait()

        for zc in zcopies:
            zc.wait()
        plsc.subcore_barrier()

        for pi, (epw, ks, nchunk) in enumerate(phases):
            msgs_hbm = msgs_refs[pi]
            idx_v = idx_v_refs[pi]
            base0 = wid * epw

            def issue(setidx, it):
                m, sm, _ = sets[setidx]
                pltpu.async_copy(
                    msgs_hbm.at[pl.ds(base0 + it * ks, ks)], m, sm)

            def process(setidx, it, issue_next):
                m, sm, sa = sets[setidx]
                pltpu.make_async_copy(
                    msgs_hbm.at[pl.ds(base0, ks)], m, sm).wait()
                if issue_next:
                    om, _osm, osa = sets[1 - setidx]

                    @pl.when(it >= 1)
                    def _wait_other_add():
                        pltpu.make_async_copy(
                            om, acc_sh.at[idx_v.at[0]], osa).wait()
                    issue(1 - setidx, it + 1)
                pltpu.async_copy(m, acc_sh.at[idx_v.at[it]], sa, add=True)

            issue(0, 0)

            def body2(k, carry):
                process(0, 2 * k, True)
                process(1, 2 * k + 1, True)
                return carry

            ndouble = (nchunk - 1) // 2
            lax.fori_loop(0, ndouble, body2, 0, unroll=False)
            for it in range(2 * ndouble, nchunk):
                process(it % 2, it, it + 1 < nchunk)
            pltpu.make_async_copy(m1, acc_sh.at[idx_v.at[0]], sa1).wait()
            pltpu.make_async_copy(m0, acc_sh.at[idx_v.at[0]], sa0).wait()

        plsc.subcore_barrier()
        pltpu.sync_copy(acc_sh.at[pl.ds(row0, ROWS_PER_TILE)],
                        agg_hbm.at[c, pl.ds(row0, ROWS_PER_TILE)])

        @pl.when(s == NS - 1)
        def _copy_tail():
            pltpu.sync_copy(
                acc_sh.at[pl.ds(NS * ROWS_PER_TILE, TAIL_ROWS)],
                agg_hbm.at[c, pl.ds(NS * ROWS_PER_TILE, TAIL_ROWS)])

    return body


def _common_chunk(epws):
    for k in range(128, 0, -8):
        if all(epw % k == 0 for epw in epws):
            return k
    raise ValueError(epws)


@functools.lru_cache(maxsize=None)
def _build_scatter(e_cnts):
    epws = [e_cnt // NW for e_cnt in e_cnts]
    ks = _common_chunk(epws)
    phases = tuple((epw, ks, epw // ks) for epw in epws)
    return pl.kernel(
        _make_scatter_body(phases),
        out_type=jax.ShapeDtypeStruct((NC, N, D_MSG), jnp.float32),
        mesh=_get_mesh(),
        scratch_types=(
            [pltpu.VMEM((nchunk, ks), jnp.int32)
             for _, ks, nchunk in phases]
            + [pltpu.VMEM((ks, D_MSG), jnp.float32)] * 2
            + [pltpu.VMEM((ZR, D_MSG), jnp.float32),
               pltpu.VMEM_SHARED((N, D_MSG), jnp.float32)]
            + [pltpu.SemaphoreType.DMA] * 5
        ),
    )


def _sc_scatter(msgs_list, ei_list):
    e_cnts = tuple(ei.shape[0] for ei in ei_list)
    ks = _common_chunk([e // NW for e in e_cnts])
    args = []
    for msgs, ei, e_cnt in zip(msgs_list, ei_list, e_cnts):
        epw = e_cnt // NW
        args += [msgs, ei.reshape(NW, epw // ks, ks)]
    return _build_scatter(e_cnts)(*args)


_NBLK = 2000
_EBLK = 2000


def _bf16_pack(even, odd):
    be = lax.bitcast_convert_type(
        even.astype(jnp.bfloat16).astype(jnp.float32), jnp.uint32)
    bo = lax.bitcast_convert_type(
        odd.astype(jnp.bfloat16).astype(jnp.float32), jnp.uint32)
    w = (be >> jnp.uint32(16)) | (bo & jnp.uint32(0xFFFF0000))
    return lax.bitcast_convert_type(w, jnp.float32)


def _bf16_unpack(packed):
    u = lax.bitcast_convert_type(packed, jnp.uint32)
    even = lax.bitcast_convert_type(u << jnp.uint32(16), jnp.float32)
    odd = lax.bitcast_convert_type(u & jnp.uint32(0xFFFF0000), jnp.float32)
    return even, odd


def _node_pre_body(x_ref, id_ref, wxie_ref, wxio_ref, wxje_ref, wxjo_ref,
                   widie_ref, widio_ref, widje_ref, widjo_ref,
                   u1x_ref, ub1_ref, pd_ref, ps_ref, xu_ref):
    x = x_ref[...]
    idn = id_ref[...]
    f32 = jnp.float32
    pd_e = (jnp.dot(x, wxie_ref[...], preferred_element_type=f32)
            + jnp.dot(idn, widie_ref[...], preferred_element_type=f32))
    pd_o = (jnp.dot(x, wxio_ref[...], preferred_element_type=f32)
            + jnp.dot(idn, widio_ref[...], preferred_element_type=f32))
    ps_e = (jnp.dot(x, wxje_ref[...], preferred_element_type=f32)
            + jnp.dot(idn, widje_ref[...], preferred_element_type=f32))
    ps_o = (jnp.dot(x, wxjo_ref[...], preferred_element_type=f32)
            + jnp.dot(idn, widjo_ref[...], preferred_element_type=f32))
    pd_ref[...] = _bf16_pack(pd_e, pd_o)
    ps_ref[...] = _bf16_pack(ps_e, ps_o)
    xu_ref[...] = (jnp.dot(x, u1x_ref[...], preferred_element_type=f32)
                   + ub1_ref[...])


def _node_precompute(x, identifiers, wxi, wxj, widi, widj, u1x, ub1):
    grid = (N // _NBLK,)
    full = lambda shape: pl.BlockSpec(shape, lambda i: (0,) * len(shape))
    return pl.pallas_call(
        _node_pre_body,
        grid=grid,
        in_specs=[
            pl.BlockSpec((_NBLK, D_IN), lambda i: (i, 0)),
            pl.BlockSpec((_NBLK, D_ID), lambda i: (i, 0)),
            full((D_IN, HD)), full((D_IN, HD)),
            full((D_IN, HD)), full((D_IN, HD)),
            full((D_ID, HD)), full((D_ID, HD)),
            full((D_ID, HD)), full((D_ID, HD)),
            full((D_IN, D_H)), full((1, D_H)),
        ],
        out_specs=[
            pl.BlockSpec((_NBLK, HD), lambda i: (i, 0)),
            pl.BlockSpec((_NBLK, HD), lambda i: (i, 0)),
            pl.BlockSpec((_NBLK, D_H), lambda i: (i, 0)),
        ],
        out_shape=[
            jax.ShapeDtypeStruct((N, HD), jnp.float32),
            jax.ShapeDtypeStruct((N, HD), jnp.float32),
            jax.ShapeDtypeStruct((N, D_H), jnp.float32),
        ],
    )(x, identifiers,
      wxi[:, 0::2], wxi[:, 1::2], wxj[:, 0::2], wxj[:, 1::2],
      widi[:, 0::2], widi[:, 1::2], widj[:, 0::2], widj[:, 1::2],
      u1x, ub1)


def _edge_mlp_body(pre_ref, ef_ref, wefe_ref, wefo_ref, b1e_ref, b1o_ref,
                   w2e_ref, w2o_ref, b2_ref, out_ref):
    f32 = jnp.float32
    ef = ef_ref[...]
    even, odd = _bf16_unpack(pre_ref[...])
    he = jnp.maximum(
        even + jnp.dot(ef, wefe_ref[...], preferred_element_type=f32)
        + b1e_ref[...], 0.0)
    ho = jnp.maximum(
        odd + jnp.dot(ef, wefo_ref[...], preferred_element_type=f32)
        + b1o_ref[...], 0.0)
    bf = jnp.bfloat16
    out_ref[...] = (jnp.dot(he.astype(bf), w2e_ref[...].astype(bf),
                            preferred_element_type=f32)
                    + jnp.dot(ho.astype(bf), w2o_ref[...].astype(bf),
                              preferred_element_type=f32)
                    + b2_ref[...])


def _edge_mlp(pre, ef, wef, b1, w2, b2):
    e_cnt = pre.shape[0]
    grid = (e_cnt // _EBLK,)
    full = lambda shape: pl.BlockSpec(shape, lambda i: (0,) * len(shape))
    return pl.pallas_call(
        _edge_mlp_body,
        grid=grid,
        in_specs=[
            pl.BlockSpec((_EBLK, HD), lambda i: (i, 0)),
            pl.BlockSpec((_EBLK, D_EF), lambda i: (i, 0)),
            full((D_EF, HD)), full((D_EF, HD)),
            full((1, HD)), full((1, HD)),
            full((HD, D_MSG)), full((HD, D_MSG)), full((1, D_MSG)),
        ],
        out_specs=pl.BlockSpec((_EBLK, D_MSG), lambda i: (i, 0)),
        out_shape=jax.ShapeDtypeStruct((e_cnt, D_MSG), jnp.float32),
    )(pre, ef, wef[:, 0::2], wef[:, 1::2], b1[:, 0::2], b1[:, 1::2],
      w2[0::2], w2[1::2], b2)


def _make_update_body(nparts):
    def _update_body(xu_ref, agg_ref, u1a_ref, w2_ref, b2_ref, out_ref):
        f32 = jnp.float32
        agg = agg_ref[0]
        for k in range(1, nparts):
            agg = agg + agg_ref[k]
        h = jnp.maximum(
            xu_ref[...]
            + jnp.dot(agg, u1a_ref[...], preferred_element_type=f32), 0.0)
        out_ref[...] = (jnp.dot(h, w2_ref[...], preferred_element_type=f32)
                        + b2_ref[...])
    return _update_body


def _update_mlp(xu, aggp, u1a, w2, b2):
    grid = (N // _NBLK,)
    full = lambda shape: pl.BlockSpec(shape, lambda i: (0,) * len(shape))
    nparts = aggp.shape[0]
    return pl.pallas_call(
        _make_update_body(nparts),
        grid=grid,
        in_specs=[
            pl.BlockSpec((_NBLK, D_H), lambda i: (i, 0)),
            pl.BlockSpec((nparts, _NBLK, D_MSG), lambda i: (0, i, 0)),
            full((D_MSG, D_H)),
            full((D_H, D_MSG)), full((1, D_MSG)),
        ],
        out_specs=pl.BlockSpec((_NBLK, D_MSG), lambda i: (i, 0)),
        out_shape=jax.ShapeDtypeStruct((N, D_MSG), jnp.float32),
    )(xu, aggp, u1a, w2, b2)


def kernel(x, edge_index, identifiers, degrees, edge_features,
           msg_W1, msg_b1, msg_W2, msg_b2,
           upd_W1, upd_b1, upd_W2, upd_b2):
    ei = edge_index[1]
    ej = edge_index[0]
    wxi = msg_W1[0:D_IN]
    wxj = msg_W1[D_IN:2 * D_IN]
    widi = msg_W1[2 * D_IN:2 * D_IN + D_ID]
    widj = msg_W1[2 * D_IN + D_ID:2 * (D_IN + D_ID)]
    wef = msg_W1[2 * (D_IN + D_ID):]
    u1x = upd_W1[0:D_IN]
    u1a = upd_W1[D_IN:]
    b1 = msg_b1.reshape(1, D_H)
    b2 = msg_b2.reshape(1, D_MSG)
    ub1 = upd_b1.reshape(1, D_H)
    ub2 = upd_b2.reshape(1, D_MSG)

    pd, ps, xu = _node_precompute(x, identifiers, wxi, wxj, widi, widj, u1x, ub1)

    msgs_list, ei_list = [], []
    lo = 0
    for es in SLICES:
        hi = lo + es
        pre_s = _sc_gather(pd, ps, ei[lo:hi], ej[lo:hi])
        msgs_list.append(
            _edge_mlp(pre_s, edge_features[lo:hi], wef, b1, msg_W2, b2))
        ei_list.append(ei[lo:hi])
        lo = hi

    aggp = _sc_scatter(msgs_list, ei_list)
    return _update_mlp(xu, aggp, u1a, upd_W2, ub2)


# speedup vs baseline: 1.1182x; 1.1182x over previous
import functools

import jax
import jax.numpy as jnp
from jax import lax
from jax.experimental import pallas as pl
from jax.experimental.pallas import tpu as pltpu
from jax.experimental.pallas import tpu_sc as plsc

N = 10000
E = 320000
D_IN = 128
D_ID = 16
D_EF = 16
D_MSG = 128
D_H = 256
HD = D_H // 2

NC = 2
NS = 16
LANES = 16
NW = NC * NS

SLICES = (192000, 128000)
ROWS_PER_TILE = 624
TAIL_ROWS = N - NS * ROWS_PER_TILE
ZR = 48


def _chunk_size(epw):
    for k in range(128, 0, -8):
        if epw % k == 0:
            return k
    raise ValueError(epw)


@functools.lru_cache(maxsize=None)
def _get_mesh():
    return plsc.VectorSubcoreMesh(core_axis_name="c", subcore_axis_name="s",
                                  num_cores=NC, num_subcores=NS)


def _make_gather_body(epw, kg, nchunk):
    nbuf = 3

    def body(pd_hbm, ps_hbm, ei_hbm, ej_hbm, pre_hbm,
             idxi_all, idxj_all, a0, b0, a1, b1, a2, b2,
             gsa0, gsb0, gsa1, gsb1, gsa2, gsb2, ss0, ss1, ss2):
        wid = lax.axis_index("s") * NC + lax.axis_index("c")
        base0 = wid * epw
        pltpu.sync_copy(ei_hbm.at[pl.ds(base0, epw)], idxi_all)
        pltpu.sync_copy(ej_hbm.at[pl.ds(base0, epw)], idxj_all)

        sets = ((a0, b0, gsa0, gsb0, ss0),
                (a1, b1, gsa1, gsb1, ss1),
                (a2, b2, gsa2, gsb2, ss2))

        def issue(setidx, it):
            a, b, gsa, gsb, _ = sets[setidx]
            off = it * kg
            pltpu.async_copy(pd_hbm.at[idxi_all.at[pl.ds(off, kg)]], a, gsa)
            pltpu.async_copy(ps_hbm.at[idxj_all.at[pl.ds(off, kg)]], b, gsb)

        def process(setidx, it, issue_next):
            a, b, gsa, gsb, ss = sets[setidx]
            nset = (setidx + 1) % nbuf
            na = sets[nset][0]
            nss = sets[nset][4]
            pltpu.make_async_copy(
                pd_hbm.at[idxi_all.at[pl.ds(0, kg)]], a, gsa).wait()
            pltpu.make_async_copy(
                ps_hbm.at[idxj_all.at[pl.ds(0, kg)]], b, gsb).wait()
            if issue_next:
                @pl.when(it >= nbuf - 1)
                def _wait_next_store():
                    pltpu.make_async_copy(
                        na, pre_hbm.at[pl.ds(base0, kg)], nss).wait()
                issue(nset, it + 1)

            def row(r, carry):
                for cc in range(HD // LANES):
                    sl = pl.ds(cc * LANES, LANES)
                    av = plsc.bitcast(a[r, sl], jnp.bfloat16)
                    bv = plsc.bitcast(b[r, sl], jnp.bfloat16)
                    a[r, sl] = plsc.bitcast(av + bv, jnp.float32)
                return carry

            lax.fori_loop(0, kg, row, 0, unroll=False)
            pltpu.async_copy(a, pre_hbm.at[pl.ds(base0 + it * kg, kg)], ss)

        issue(0, 0)
        ntriple = (nchunk - 2) // nbuf

        def body3(k, carry):
            process(0, 3 * k, True)
            process(1, 3 * k + 1, True)
            process(2, 3 * k + 2, True)
            return carry

        lax.fori_loop(0, ntriple, body3, 0, unroll=False)
        for it in range(nbuf * ntriple, nchunk):
            process(it % nbuf, it, it + 1 < nchunk)
        for a, _b, _gsa, _gsb, ss in sets:
            pltpu.make_async_copy(a, pre_hbm.at[pl.ds(base0, kg)], ss).wait()

    return body


@functools.lru_cache(maxsize=None)
def _build_gather(e_cnt):
    epw = e_cnt // NW
    kg = _chunk_size(epw)
    nchunk = epw // kg
    return pl.kernel(
        _make_gather_body(epw, kg, nchunk),
        out_type=jax.ShapeDtypeStruct((e_cnt, HD), jnp.float32),
        mesh=_get_mesh(),
        compiler_params=pltpu.CompilerParams(needs_layout_passes=False,
                                             use_tc_tiling_on_sc=True),
        scratch_types=(
            [pltpu.VMEM((epw,), jnp.int32)] * 2
            + [pltpu.VMEM((kg, HD), jnp.float32)] * 6
            + [pltpu.SemaphoreType.DMA] * 9
        ),
    )


def _sc_gather(pd, ps, ei, ej):
    return _build_gather(ei.shape[0])(pd, ps, ei, ej)


def _make_scatter_body(phases):
    def body(*refs):
        nph = len(phases)
        msgs_refs = refs[0:2 * nph:2]
        idx_hbm_refs = refs[1:2 * nph:2]
        agg_hbm = refs[2 * nph]
        idx_v_refs = refs[2 * nph + 1:3 * nph + 1]
        m0, m1, z_v, acc_sh, sm0, sm1, sa0, sa1, zsem = refs[3 * nph + 1:]
        c = lax.axis_index("c")
        s = lax.axis_index("s")
        wid = c * NS + s
        sets = ((m0, sm0, sa0), (m1, sm1, sa1))

        for ih, iv in zip(idx_hbm_refs, idx_v_refs):
            pltpu.sync_copy(ih.at[wid], iv)
        zero = jnp.zeros((LANES,), jnp.float32)

        def zrow(r, carry):
            for cc in range(D_MSG // LANES):
                z_v[r, pl.ds(cc * LANES, LANES)] = zero
            return carry

        lax.fori_loop(0, ZR, zrow, 0, unroll=False)
        row0 = s * ROWS_PER_TILE
        zcopies = [
            pltpu.async_copy(z_v, acc_sh.at[pl.ds(row0 + k * ZR, ZR)], zsem)
            for k in range(ROWS_PER_TILE // ZR)
        ]

        @pl.when(s == NS - 1)
        def _zero_tail():
            pltpu.async_copy(z_v.at[pl.ds(0, TAIL_ROWS)],
                             acc_sh.at[pl.ds(NS * ROWS_PER_TILE, TAIL_ROWS)],
                             zsem).wait()

        for zc in zcopies:
            zc.wait()
        plsc.subcore_barrier()

        for pi, (epw, ks, nchunk) in enumerate(phases):
            msgs_hbm = msgs_refs[pi]
            idx_v = idx_v_refs[pi]
            base0 = wid * epw

            def issue(setidx, it):
                m, sm, _ = sets[setidx]
                pltpu.async_copy(
                    msgs_hbm.at[pl.ds(base0 + it * ks, ks)], m, sm)

            def process(setidx, it, issue_next):
                m, sm, sa = sets[setidx]
                pltpu.make_async_copy(
                    msgs_hbm.at[pl.ds(base0, ks)], m, sm).wait()
                if issue_next:
                    om, _osm, osa = sets[1 - setidx]

                    @pl.when(it >= 1)
                    def _wait_other_add():
                        pltpu.make_async_copy(
                            om, acc_sh.at[idx_v.at[0]], osa).wait()
                    issue(1 - setidx, it + 1)
                pltpu.async_copy(m, acc_sh.at[idx_v.at[it]], sa, add=True)

            issue(0, 0)

            def body2(k, carry):
                process(0, 2 * k, True)
                process(1, 2 * k + 1, True)
                return carry

            ndouble = (nchunk - 1) // 2
            lax.fori_loop(0, ndouble, body2, 0, unroll=False)
            for it in range(2 * ndouble, nchunk):
                process(it % 2, it, it + 1 < nchunk)
            pltpu.make_async_copy(m1, acc_sh.at[idx_v.at[0]], sa1).wait()
            pltpu.make_async_copy(m0, acc_sh.at[idx_v.at[0]], sa0).wait()

        plsc.subcore_barrier()
        pltpu.sync_copy(acc_sh.at[pl.ds(row0, ROWS_PER_TILE)],
                        agg_hbm.at[c, pl.ds(row0, ROWS_PER_TILE)])

        @pl.when(s == NS - 1)
        def _copy_tail():
            pltpu.sync_copy(
                acc_sh.at[pl.ds(NS * ROWS_PER_TILE, TAIL_ROWS)],
                agg_hbm.at[c, pl.ds(NS * ROWS_PER_TILE, TAIL_ROWS)])

    return body


def _common_chunk(epws):
    for k in range(128, 0, -8):
        if all(epw % k == 0 for epw in epws):
            return k
    raise ValueError(epws)


@functools.lru_cache(maxsize=None)
def _build_scatter(e_cnts):
    epws = [e_cnt // NW for e_cnt in e_cnts]
    ks = _common_chunk(epws)
    phases = tuple((epw, ks, epw // ks) for epw in epws)
    return pl.kernel(
        _make_scatter_body(phases),
        out_type=jax.ShapeDtypeStruct((NC, N, D_MSG), jnp.float32),
        mesh=_get_mesh(),
        scratch_types=(
            [pltpu.VMEM((nchunk, ks), jnp.int32)
             for _, ks, nchunk in phases]
            + [pltpu.VMEM((ks, D_MSG), jnp.float32)] * 2
            + [pltpu.VMEM((ZR, D_MSG), jnp.float32),
               pltpu.VMEM_SHARED((N, D_MSG), jnp.float32)]
            + [pltpu.SemaphoreType.DMA] * 5
        ),
    )


def _sc_scatter(msgs_list, ei_list):
    e_cnts = tuple(ei.shape[0] for ei in ei_list)
    ks = _common_chunk([e // NW for e in e_cnts])
    args = []
    for msgs, ei, e_cnt in zip(msgs_list, ei_list, e_cnts):
        epw = e_cnt // NW
        args += [msgs, ei.reshape(NW, epw // ks, ks)]
    return _build_scatter(e_cnts)(*args)


_NBLK = 2000
_EBLK = 1280


def _bf16_pack(even, odd):
    be = lax.bitcast_convert_type(
        even.astype(jnp.bfloat16).astype(jnp.float32), jnp.uint32)
    bo = lax.bitcast_convert_type(
        odd.astype(jnp.bfloat16).astype(jnp.float32), jnp.uint32)
    w = (be >> jnp.uint32(16)) | (bo & jnp.uint32(0xFFFF0000))
    return lax.bitcast_convert_type(w, jnp.float32)


def _bf16_unpack(packed):
    u = lax.bitcast_convert_type(packed, jnp.uint32)
    even = lax.bitcast_convert_type(u << jnp.uint32(16), jnp.float32)
    odd = lax.bitcast_convert_type(u & jnp.uint32(0xFFFF0000), jnp.float32)
    return even, odd


def _dot_t(lhs_t, rhs):
    return lax.dot_general(lhs_t, rhs, (((0,), (0,)), ((), ())),
                           preferred_element_type=jnp.float32)


def _node_pre_body(x_ref, idt_ref, wxie_ref, wxio_ref, wxje_ref, wxjo_ref,
                   widie_ref, widio_ref, widje_ref, widjo_ref,
                   u1x_ref, ub1_ref, pd_ref, ps_ref, xu_ref):
    x = x_ref[...]
    idn = idt_ref[...]
    f32 = jnp.float32
    pd_e = (jnp.dot(x, wxie_ref[...], preferred_element_type=f32)
            + jnp.dot(idn, widie_ref[...], preferred_element_type=f32))
    pd_o = (jnp.dot(x, wxio_ref[...], preferred_element_type=f32)
            + jnp.dot(idn, widio_ref[...], preferred_element_type=f32))
    ps_e = (jnp.dot(x, wxje_ref[...], preferred_element_type=f32)
            + jnp.dot(idn, widje_ref[...], preferred_element_type=f32))
    ps_o = (jnp.dot(x, wxjo_ref[...], preferred_element_type=f32)
            + jnp.dot(idn, widjo_ref[...], preferred_element_type=f32))
    pd_ref[...] = _bf16_pack(pd_e, pd_o)
    ps_ref[...] = _bf16_pack(ps_e, ps_o)
    xu_ref[...] = (jnp.dot(x, u1x_ref[...], preferred_element_type=f32)
                   + ub1_ref[...])


def _node_precompute(x, id_t, wxi, wxj, widi, widj, u1x, ub1):
    grid = (N // _NBLK,)
    full = lambda shape: pl.BlockSpec(shape, lambda i: (0,) * len(shape))
    return pl.pallas_call(
        _node_pre_body,
        grid=grid,
        in_specs=[
            pl.BlockSpec((_NBLK, D_IN), lambda i: (i, 0)),
            pl.BlockSpec((_NBLK, D_ID), lambda i: (i, 0)),
            full((D_IN, HD)), full((D_IN, HD)),
            full((D_IN, HD)), full((D_IN, HD)),
            full((D_ID, HD)), full((D_ID, HD)),
            full((D_ID, HD)), full((D_ID, HD)),
            full((D_IN, D_H)), full((1, D_H)),
        ],
        out_specs=[
            pl.BlockSpec((_NBLK, HD), lambda i: (i, 0)),
            pl.BlockSpec((_NBLK, HD), lambda i: (i, 0)),
            pl.BlockSpec((_NBLK, D_H), lambda i: (i, 0)),
        ],
        out_shape=[
            jax.ShapeDtypeStruct((N, HD), jnp.float32),
            jax.ShapeDtypeStruct((N, HD), jnp.float32),
            jax.ShapeDtypeStruct((N, D_H), jnp.float32),
        ],
    )(x, id_t,
      wxi[:, 0::2], wxi[:, 1::2], wxj[:, 0::2], wxj[:, 1::2],
      widi[:, 0::2], widi[:, 1::2], widj[:, 0::2], widj[:, 1::2],
      u1x, ub1)


def _edge_mlp_body(pre_ref, eft_ref, wefe_ref, wefo_ref, b1e_ref, b1o_ref,
                   w2e_ref, w2o_ref, b2_ref, out_ref):
    eft = eft_ref[...]
    even, odd = _bf16_unpack(pre_ref[...])
    he = jnp.maximum(even + _dot_t(eft, wefe_ref[...]) + b1e_ref[...], 0.0)
    ho = jnp.maximum(odd + _dot_t(eft, wefo_ref[...]) + b1o_ref[...], 0.0)
    f32 = jnp.float32
    bf = jnp.bfloat16
    out_ref[...] = (jnp.dot(he.astype(bf), w2e_ref[...].astype(bf),
                            preferred_element_type=f32)
                    + jnp.dot(ho.astype(bf), w2o_ref[...].astype(bf),
                              preferred_element_type=f32)
                    + b2_ref[...])


def _edge_mlp(pre, ef_t, eblk0, wef, b1, w2, b2):
    e_cnt = pre.shape[0]
    grid = (e_cnt // _EBLK,)
    full = lambda shape: pl.BlockSpec(shape, lambda i: (0,) * len(shape))
    return pl.pallas_call(
        _edge_mlp_body,
        grid=grid,
        in_specs=[
            pl.BlockSpec((_EBLK, HD), lambda i: (i, 0)),
            pl.BlockSpec((D_EF, _EBLK), lambda i: (0, i + eblk0)),
            full((D_EF, HD)), full((D_EF, HD)),
            full((1, HD)), full((1, HD)),
            full((HD, D_MSG)), full((HD, D_MSG)), full((1, D_MSG)),
        ],
        out_specs=pl.BlockSpec((_EBLK, D_MSG), lambda i: (i, 0)),
        out_shape=jax.ShapeDtypeStruct((e_cnt, D_MSG), jnp.float32),
    )(pre, ef_t, wef[:, 0::2], wef[:, 1::2], b1[:, 0::2], b1[:, 1::2],
      w2[0::2], w2[1::2], b2)


def _make_update_body(nparts):
    def _update_body(xu_ref, agg_ref, u1a_ref, w2_ref, b2_ref, out_ref):
        f32 = jnp.float32
        agg = agg_ref[0]
        for k in range(1, nparts):
            agg = agg + agg_ref[k]
        h = jnp.maximum(
            xu_ref[...]
            + jnp.dot(agg, u1a_ref[...], preferred_element_type=f32), 0.0)
        out_ref[...] = (jnp.dot(h, w2_ref[...], preferred_element_type=f32)
                        + b2_ref[...])
    return _update_body


def _update_mlp(xu, aggp, u1a, w2, b2):
    grid = (N // _NBLK,)
    full = lambda shape: pl.BlockSpec(shape, lambda i: (0,) * len(shape))
    nparts = aggp.shape[0]
    return pl.pallas_call(
        _make_update_body(nparts),
        grid=grid,
        in_specs=[
            pl.BlockSpec((_NBLK, D_H), lambda i: (i, 0)),
            pl.BlockSpec((nparts, _NBLK, D_MSG), lambda i: (0, i, 0)),
            full((D_MSG, D_H)),
            full((D_H, D_MSG)), full((1, D_MSG)),
        ],
        out_specs=pl.BlockSpec((_NBLK, D_MSG), lambda i: (i, 0)),
        out_shape=jax.ShapeDtypeStruct((N, D_MSG), jnp.float32),
    )(xu, aggp, u1a, w2, b2)


def kernel(x, edge_index, identifiers, degrees, edge_features,
           msg_W1, msg_b1, msg_W2, msg_b2,
           upd_W1, upd_b1, upd_W2, upd_b2):
    ei = edge_index[1]
    ej = edge_index[0]
    wxi = msg_W1[0:D_IN]
    wxj = msg_W1[D_IN:2 * D_IN]
    widi = msg_W1[2 * D_IN:2 * D_IN + D_ID]
    widj = msg_W1[2 * D_IN + D_ID:2 * (D_IN + D_ID)]
    wef = msg_W1[2 * (D_IN + D_ID):]
    u1x = upd_W1[0:D_IN]
    u1a = upd_W1[D_IN:]
    b1 = msg_b1.reshape(1, D_H)
    b2 = msg_b2.reshape(1, D_MSG)
    ub1 = upd_b1.reshape(1, D_H)
    ub2 = upd_b2.reshape(1, D_MSG)

    pd, ps, xu = _node_precompute(x, identifiers, wxi, wxj, widi, widj,
                                  u1x, ub1)

    ef_t = edge_features.T
    msgs_list, ei_list = [], []
    lo = 0
    for es in SLICES:
        hi = lo + es
        pre_s = _sc_gather(pd, ps, ei[lo:hi], ej[lo:hi])
        msgs_list.append(
            _edge_mlp(pre_s, ef_t, lo // _EBLK, wef, b1, msg_W2, b2))
        ei_list.append(ei[lo:hi])
        lo = hi

    aggp = _sc_scatter(msgs_list, ei_list)
    return _update_mlp(xu, aggp, u1a, upd_W2, ub2)
